# baseline (speedup 1.0000x reference)
import functools

import jax
import jax.numpy as jnp
from jax import lax
from jax.experimental import pallas as pl
from jax.experimental.pallas import tpu as pltpu
from jax.experimental.pallas import tpu_sc as plsc

B, N, D, S, DOUT = 2, 65536, 16, 8, 16
NS = N * S
DH = 2 * D
LW = 128

NBP = 4096


def _tc_project_body(x_ref, w1_ref, b1_ref, tp_ref):
    xa = x_ref[...]
    w1 = w1_ref[...]
    pj = jnp.dot(xa, w1[D:2 * D], preferred_element_type=jnp.float32)
    pk = jnp.dot(xa, w1[2 * D:], preferred_element_type=jnp.float32)
    pi = jnp.dot(xa, w1[:D], preferred_element_type=jnp.float32) + b1_ref[...]
    z = jnp.zeros((NBP, DH), jnp.float32)
    tp_ref[...] = jnp.concatenate([pj, pk, pi, z], axis=1)


_tc_project = pl.pallas_call(
    _tc_project_body,
    grid=((B * N) // NBP,),
    in_specs=[
        pl.BlockSpec((NBP, D), lambda i: (i, 0)),
        pl.BlockSpec((3 * D, DH), lambda i: (0, 0)),
        pl.BlockSpec((1, DH), lambda i: (0, 0)),
    ],
    out_specs=pl.BlockSpec((NBP, LW), lambda i: (i, 0)),
    out_shape=jax.ShapeDtypeStruct((B * N, LW), jnp.float32),
)

NC, NSUB = 2, 16
NW = NC * NSUB
PW = NS // NW
GI = 128
QO = 4
CO = GI * QO
CHUNKS = PW // CO
NCH = NS // CO
GROWS = NS // QO


PWC = NS // NSUB
CHUNKS1 = PWC // CO


def _make_sc_body(t):
    lo = t * DH

    def body(tp_hbm, jk_hbm, g_hbm, idx_v, pk_v, pb_v, sem):
        wid = lax.axis_index("s")

        for b in range(B):
            tb = t * B + b

            def chunk_body(g, _, tb=tb, b=b):
                ch = wid * CHUNKS1 + g
                r0 = (tb * NCH + ch) * QO
                pltpu.sync_copy(jk_hbm.at[pl.ds(r0, QO)], idx_v)
                cps = [
                    pltpu.async_copy(
                        tp_hbm.at[idx_v.at[q]],
                        pk_v.at[pl.ds(q * GI, GI)],
                        sem,
                    )
                    for q in range(QO)
                ]
                for cp in cps:
                    cp.wait()
                for q in range(QO):
                    for r in range(GI):
                        for h in range(2):
                            pb_v[r, pl.ds(q * DH + h * 16, 16)] = (
                                pk_v[q * GI + r, pl.ds(lo + h * 16, 16)]
                            )
                o0 = b * GROWS + ch * GI
                pltpu.sync_copy(pb_v, g_hbm.at[pl.ds(o0, GI)])
                return 0

            lax.fori_loop(0, CHUNKS1, chunk_body, 0, unroll=False)

    return body


@functools.cache
def _sc_gather(t):
    return pl.kernel(
        _make_sc_body(t),
        out_type=jax.ShapeDtypeStruct((B * GROWS, LW), jnp.float32),
        mesh=plsc.VectorSubcoreMesh(
            core_axis_name="c", subcore_axis_name="s", num_cores=1
        ),
        scratch_types=[
            pltpu.VMEM((QO, GI), jnp.int32),
            pltpu.VMEM((CO, LW), jnp.float32),
            pltpu.VMEM((GI, LW), jnp.float32),
            pltpu.SemaphoreType.DMA,
        ],
    )


NB = 2048
GRID = (B * N) // NB
RPN = S // QO
_INV_SQRT2 = 0.7071067811865476


def _tc_combine_body(tp_ref, gj_ref, gk_ref, w2r_ref, b2_ref, out_ref):
    a = tp_ref[...][:, 2 * DH:3 * DH]
    a4 = jnp.concatenate([a] * QO, axis=1)
    a8 = jnp.repeat(a4, RPN, axis=0)
    t = gj_ref[...] + gk_ref[...] + a8
    h = 0.5 * t * (1.0 + lax.erf(t * _INV_SQRT2))
    o = jnp.dot(h, w2r_ref[...], preferred_element_type=jnp.float32)
    o = o.reshape(NB, RPN, DOUT).sum(axis=1)
    out_ref[...] = o + b2_ref[...]


_tc_combine = pl.pallas_call(
    _tc_combine_body,
    grid=(GRID,),
    in_specs=[
        pl.BlockSpec((NB, LW), lambda i: (i, 0)),
        pl.BlockSpec((RPN * NB, LW), lambda i: (i, 0)),
        pl.BlockSpec((RPN * NB, LW), lambda i: (i, 0)),
        pl.BlockSpec((LW, DOUT), lambda i: (0, 0)),
        pl.BlockSpec((1, DOUT), lambda i: (0, 0)),
    ],
    out_specs=pl.BlockSpec((NB, DOUT), lambda i: (i, 0)),
    out_shape=jax.ShapeDtypeStruct((B * N, DOUT), jnp.float32),
)


def kernel(x, j_idx, k_idx, W1, b1, W2, b2):
    off = (jnp.arange(B, dtype=jnp.int32) * N)[None, :, None]
    jk = (
        (jnp.stack([j_idx.reshape(B, NS), k_idx.reshape(B, NS)]) + off)
        .reshape(2, B, NCH, GI, QO)
        .swapaxes(3, 4)
        .reshape(2 * B * NCH * QO, GI)
    )
    tp = _tc_project(x.reshape(B * N, D), W1, b1.reshape(1, DH))
    gj = _sc_gather(0)(tp, jk)
    gk = _sc_gather(1)(tp, jk)
    w2r = jnp.concatenate([W2] * QO, axis=0) / S
    out = _tc_combine(tp, gj, gk, w2r, b2.reshape(1, DOUT))
    return out.reshape(B, N, DOUT)


# speedup vs baseline: 1.4999x; 1.4999x over previous
import functools

import jax
import jax.numpy as jnp
from jax import lax
from jax.experimental import pallas as pl
from jax.experimental.pallas import tpu as pltpu
from jax.experimental.pallas import tpu_sc as plsc

B, N, D, S, DOUT = 2, 65536, 16, 8, 16
NS = N * S
DH = 2 * D
LW = 128

NBP = 4096


def _tc_project_body(x_ref, w1_ref, b1_ref, tp_ref):
    xa = x_ref[...]
    w1 = w1_ref[...]
    pj = jnp.dot(xa, w1[D:2 * D], preferred_element_type=jnp.float32)
    pk = jnp.dot(xa, w1[2 * D:], preferred_element_type=jnp.float32)
    pi = jnp.dot(xa, w1[:D], preferred_element_type=jnp.float32) + b1_ref[...]
    z = jnp.zeros((NBP, DH), jnp.float32)
    tp_ref[...] = jnp.concatenate([pj, pk, pi, z], axis=1)


_tc_project = pl.pallas_call(
    _tc_project_body,
    grid=((B * N) // NBP,),
    in_specs=[
        pl.BlockSpec((NBP, D), lambda i: (i, 0)),
        pl.BlockSpec((3 * D, DH), lambda i: (0, 0)),
        pl.BlockSpec((1, DH), lambda i: (0, 0)),
    ],
    out_specs=pl.BlockSpec((NBP, LW), lambda i: (i, 0)),
    out_shape=jax.ShapeDtypeStruct((B * N, LW), jnp.float32),
)

NC, NSUB = 2, 16
NW = NC * NSUB
PW = NS // NW
GI = 128
QO = 4
CO = GI * QO
CHUNKS = PW // CO
NCH = NS // CO
GROWS = NS // QO


IDXR = NS // GI
OPW = IDXR // NW
OPC = 8
ROWS_PER_OP = GI // QO


def _sc_gather_body(tp_hbm, jk_hbm, g_hbm, idx_v, pk_v, pb_v,
                    sg0, sg1, sw0, sw1):
    cid = lax.axis_index("c")
    sid = lax.axis_index("s")
    wid = sid * NC + cid
    sg = (sg0, sg1)
    sw = (sw0, sw1)

    def gather(q, row):
        return pltpu.async_copy(
            tp_hbm.at[idx_v.at[row]],
            pk_v.at[pl.ds((q % 2) * GI, GI)],
            sg[q % 2],
        )

    for t in range(2):
        lo = t * DH

        def chunk_body(g2, _, t=t, lo=lo):
            b = g2 // (OPW // OPC)
            g = g2 % (OPW // OPC)
            tb = t * B + b
            op0 = wid * OPW + g * OPC
            r0 = tb * IDXR + op0
            pltpu.sync_copy(jk_hbm.at[pl.ds(r0, OPC)], idx_v)
            writes = [None, None]
            cp = gather(0, 0)
            for q in range(OPC):
                nxt = gather(q + 1, q + 1) if q + 1 < OPC else None
                cp.wait()
                if writes[q % 2] is not None:
                    writes[q % 2].wait()
                pkb = (q % 2) * GI
                for i in range(GI):
                    for h in range(2):
                        pb_v[(q % 2) * ROWS_PER_OP + i // QO,
                             pl.ds((i % QO) * DH + h * 16, 16)] = (
                            pk_v[pkb + i, pl.ds(lo + h * 16, 16)]
                        )
                o0 = tb * GROWS + (op0 + q) * ROWS_PER_OP
                writes[q % 2] = pltpu.async_copy(
                    pb_v.at[pl.ds((q % 2) * ROWS_PER_OP, ROWS_PER_OP)],
                    g_hbm.at[pl.ds(o0, ROWS_PER_OP)],
                    sw[q % 2],
                )
                cp = nxt
            for w in writes:
                if w is not None:
                    w.wait()
            return 0

        lax.fori_loop(0, B * (OPW // OPC), chunk_body, 0, unroll=False)


@functools.cache
def _sc_gather():
    return pl.kernel(
        _sc_gather_body,
        out_type=jax.ShapeDtypeStruct((2 * B * GROWS, LW), jnp.float32),
        mesh=plsc.VectorSubcoreMesh(core_axis_name="c", subcore_axis_name="s"),
        scratch_types=[
            pltpu.VMEM((OPC, GI), jnp.int32),
            pltpu.VMEM((2 * GI, LW), jnp.float32),
            pltpu.VMEM((2 * ROWS_PER_OP, LW), jnp.float32),
            pltpu.SemaphoreType.DMA,
            pltpu.SemaphoreType.DMA,
            pltpu.SemaphoreType.DMA,
            pltpu.SemaphoreType.DMA,
        ],
    )


NB = 2048
GRID = (B * N) // NB
RPN = S // QO
_INV_SQRT2 = 0.7071067811865476


def _tc_combine_body(tp_ref, gj_ref, gk_ref, w2r_ref, b2_ref, out_ref):
    a = tp_ref[...][:, 2 * DH:3 * DH]
    a4 = jnp.concatenate([a] * QO, axis=1)
    a8 = jnp.repeat(a4, RPN, axis=0)
    t = gj_ref[...] + gk_ref[...] + a8
    h = 0.5 * t * (1.0 + lax.erf(t * _INV_SQRT2))
    o = jnp.dot(h, w2r_ref[...], preferred_element_type=jnp.float32)
    o = o.reshape(NB, RPN, DOUT).sum(axis=1)
    out_ref[...] = o + b2_ref[...]


_tc_combine = pl.pallas_call(
    _tc_combine_body,
    grid=(GRID,),
    in_specs=[
        pl.BlockSpec((NB, LW), lambda i: (i, 0)),
        pl.BlockSpec((RPN * NB, LW), lambda i: (i, 0)),
        pl.BlockSpec((RPN * NB, LW), lambda i: (i, 0)),
        pl.BlockSpec((LW, DOUT), lambda i: (0, 0)),
        pl.BlockSpec((1, DOUT), lambda i: (0, 0)),
    ],
    out_specs=pl.BlockSpec((NB, DOUT), lambda i: (i, 0)),
    out_shape=jax.ShapeDtypeStruct((B * N, DOUT), jnp.float32),
)


def kernel(x, j_idx, k_idx, W1, b1, W2, b2):
    off = (jnp.arange(B, dtype=jnp.int32) * N)[None, :, None]
    jk = (
        (jnp.stack([j_idx.reshape(B, NS), k_idx.reshape(B, NS)]) + off)
        .reshape(2 * B * IDXR, GI)
    )
    tp = _tc_project(x.reshape(B * N, D), W1, b1.reshape(1, DH))
    g = _sc_gather()(tp, jk)
    gj = g[: B * GROWS]
    gk = g[B * GROWS:]
    w2r = jnp.concatenate([W2] * QO, axis=0) / S
    out = _tc_combine(tp, gj, gk, w2r, b2.reshape(1, DOUT))
    return out.reshape(B, N, DOUT)


# speedup vs baseline: 1.7672x; 1.1782x over previous
import functools

import jax
import jax.numpy as jnp
from jax import lax
from jax.experimental import pallas as pl
from jax.experimental.pallas import tpu as pltpu
from jax.experimental.pallas import tpu_sc as plsc

B, N, D, S, DOUT = 2, 65536, 16, 8, 16
NS = N * S
DH = 2 * D
LW = 128

NBP = 4096


def _tc_project_body(x_ref, w1_ref, b1_ref, tp_ref):
    xa = x_ref[...]
    w1 = w1_ref[...]
    pj = jnp.dot(xa, w1[D:2 * D], preferred_element_type=jnp.float32)
    pk = jnp.dot(xa, w1[2 * D:], preferred_element_type=jnp.float32)
    pi = jnp.dot(xa, w1[:D], preferred_element_type=jnp.float32) + b1_ref[...]
    z = jnp.zeros((NBP, DH), jnp.float32)
    tp_ref[...] = jnp.concatenate([pj, pk, pi, z], axis=1)


_tc_project = pl.pallas_call(
    _tc_project_body,
    grid=((B * N) // NBP,),
    in_specs=[
        pl.BlockSpec((NBP, D), lambda i: (i, 0)),
        pl.BlockSpec((3 * D, DH), lambda i: (0, 0)),
        pl.BlockSpec((1, DH), lambda i: (0, 0)),
    ],
    out_specs=pl.BlockSpec((NBP, LW), lambda i: (i, 0)),
    out_shape=jax.ShapeDtypeStruct((B * N, LW), jnp.float32),
)

NC, NSUB = 2, 16
NW = NC * NSUB
PW = NS // NW
GI = 128
QO = 4
CO = GI * QO
CHUNKS = PW // CO
NCH = NS // CO
GROWS = NS // QO


IDXR = NS // GI
OPW = IDXR // NW
OPC = 8
ROWS_PER_OP = GI // QO


def _sc_gather_body(tp_hbm, j_hbm, k_hbm, g_hbm, idx_v, pk_v, pb_v,
                    sg0, sg1, sw0, sw1):
    cid = lax.axis_index("c")
    sid = lax.axis_index("s")
    wid = sid * NC + cid
    sg = (sg0, sg1)
    sw = (sw0, sw1)

    def gather(q, row):
        return pltpu.async_copy(
            tp_hbm.at[idx_v.at[row]],
            pk_v.at[pl.ds((q % 2) * GI, GI)],
            sg[q % 2],
        )

    for t, idx_hbm in ((0, j_hbm), (1, k_hbm)):
        lo = t * DH

        def chunk_body(g2, _, t=t, lo=lo, idx_hbm=idx_hbm):
            b = g2 // (OPW // OPC)
            g = g2 % (OPW // OPC)
            tb = t * B + b
            op0 = wid * OPW + g * OPC
            r0 = b * IDXR + op0
            pltpu.sync_copy(idx_hbm.at[pl.ds(r0, OPC)], idx_v)
            writes = [None, None]
            cp = gather(0, 0)
            for q in range(OPC):
                nxt = gather(q + 1, q + 1) if q + 1 < OPC else None
                cp.wait()
                if writes[q % 2] is not None:
                    writes[q % 2].wait()
                pkb = (q % 2) * GI
                for i in range(GI):
                    for h in range(2):
                        pb_v[(q % 2) * ROWS_PER_OP + i // QO,
                             pl.ds((i % QO) * DH + h * 16, 16)] = (
                            pk_v[pkb + i, pl.ds(lo + h * 16, 16)]
                        )
                o0 = tb * GROWS + (op0 + q) * ROWS_PER_OP
                writes[q % 2] = pltpu.async_copy(
                    pb_v.at[pl.ds((q % 2) * ROWS_PER_OP, ROWS_PER_OP)],
                    g_hbm.at[pl.ds(o0, ROWS_PER_OP)],
                    sw[q % 2],
                )
                cp = nxt
            for w in writes:
                if w is not None:
                    w.wait()
            return 0

        lax.fori_loop(0, B * (OPW // OPC), chunk_body, 0, unroll=False)


@functools.cache
def _sc_gather():
    return pl.kernel(
        _sc_gather_body,
        out_type=jax.ShapeDtypeStruct((2 * B * GROWS, LW), jnp.float32),
        mesh=plsc.VectorSubcoreMesh(core_axis_name="c", subcore_axis_name="s"),
        scratch_types=[
            pltpu.VMEM((OPC, GI), jnp.int32),
            pltpu.VMEM((2 * GI, LW), jnp.float32),
            pltpu.VMEM((2 * ROWS_PER_OP, LW), jnp.float32),
            pltpu.SemaphoreType.DMA,
            pltpu.SemaphoreType.DMA,
            pltpu.SemaphoreType.DMA,
            pltpu.SemaphoreType.DMA,
        ],
    )


NB = 1024
GRID = (B * N) // NB
RPN = S // QO
_INV_SQRT2 = 0.7071067811865476


def _tc_combine_body(tp_ref, gj_ref, gk_ref, w2r_ref, b2_ref, out_ref):
    a = tp_ref[...][:, 2 * DH:3 * DH]
    a4 = jnp.concatenate([a] * QO, axis=1)
    a8 = jnp.repeat(a4, RPN, axis=0)
    t = gj_ref[...] + gk_ref[...] + a8
    h = 0.5 * t * (1.0 + lax.erf(t * _INV_SQRT2))
    o = jnp.dot(h, w2r_ref[...], preferred_element_type=jnp.float32)
    o = o.reshape(NB, RPN, DOUT).sum(axis=1)
    out_ref[...] = o + b2_ref[...]


_KOFF = (B * GROWS) // (RPN * NB)

_tc_combine = pl.pallas_call(
    _tc_combine_body,
    grid=(GRID,),
    in_specs=[
        pl.BlockSpec((NB, LW), lambda i: (i, 0)),
        pl.BlockSpec((RPN * NB, LW), lambda i: (i, 0)),
        pl.BlockSpec((RPN * NB, LW), lambda i: (_KOFF + i, 0)),
        pl.BlockSpec((LW, DOUT), lambda i: (0, 0)),
        pl.BlockSpec((1, DOUT), lambda i: (0, 0)),
    ],
    out_specs=pl.BlockSpec((NB, DOUT), lambda i: (i, 0)),
    out_shape=jax.ShapeDtypeStruct((B * N, DOUT), jnp.float32),
)


def kernel(x, j_idx, k_idx, W1, b1, W2, b2):
    off = (jnp.arange(B, dtype=jnp.int32) * N)[:, None]
    jr = (j_idx.reshape(B, NS) + off).reshape(B * IDXR, GI)
    kr = (k_idx.reshape(B, NS) + off).reshape(B * IDXR, GI)
    tp = _tc_project(x.reshape(B * N, D), W1, b1.reshape(1, DH))
    g = _sc_gather()(tp, jr, kr)
    w2r = jnp.concatenate([W2] * QO, axis=0) / S
    out = _tc_combine(tp, g, g, w2r, b2.reshape(1, DOUT))
    return out.reshape(B, N, DOUT)


# speedup vs baseline: 1.9464x; 1.1014x over previous
import functools

import jax
import jax.numpy as jnp
from jax import lax
from jax.experimental import pallas as pl
from jax.experimental.pallas import tpu as pltpu
from jax.experimental.pallas import tpu_sc as plsc

B, N, D, S, DOUT = 2, 65536, 16, 8, 16
NS = N * S
DH = 2 * D
LW = 128

NBP = 4096


def _tc_project_body(x_ref, w1_ref, b1_ref, tp_ref):
    xa = x_ref[...]
    w1 = w1_ref[...]
    pj = jnp.dot(xa, w1[D:2 * D], preferred_element_type=jnp.float32)
    pk = jnp.dot(xa, w1[2 * D:], preferred_element_type=jnp.float32)
    pi = jnp.dot(xa, w1[:D], preferred_element_type=jnp.float32) + b1_ref[...]
    z = jnp.zeros((NBP, DH), jnp.float32)
    tp_ref[...] = jnp.concatenate([pj, pk, pi, z], axis=1)


_tc_project = pl.pallas_call(
    _tc_project_body,
    grid=((B * N) // NBP,),
    in_specs=[
        pl.BlockSpec((NBP, D), lambda i: (i, 0)),
        pl.BlockSpec((3 * D, DH), lambda i: (0, 0)),
        pl.BlockSpec((1, DH), lambda i: (0, 0)),
    ],
    out_specs=pl.BlockSpec((NBP, LW), lambda i: (i, 0)),
    out_shape=jax.ShapeDtypeStruct((B * N, LW), jnp.float32),
)

NC, NSUB = 2, 16
NW = NC * NSUB
PW = NS // NW
GI = 128
QO = 4
CO = GI * QO
CHUNKS = PW // CO
NCH = NS // CO
GROWS = NS // QO


IDXR = NS // GI
OPW = IDXR // NW
OPC = 8
SPR = S
ROWS_PER_OP = GI // SPR
GROWS2 = NS // SPR
WPS = DH // 2


def _sc_gather_body(tp_hbm, j_hbm, k_hbm, g_hbm, idx_v, pk_v, pb_v,
                    sg0, sg1, sw0, sw1):
    cid = lax.axis_index("c")
    sid = lax.axis_index("s")
    wid = sid * NC + cid
    sg = (sg0, sg1)
    sw = (sw0, sw1)

    def gather(q, row):
        return pltpu.async_copy(
            tp_hbm.at[idx_v.at[row]],
            pk_v.at[pl.ds((q % 2) * GI, GI)],
            sg[q % 2],
        )

    for t, idx_hbm in ((0, j_hbm), (1, k_hbm)):
        lo = t * DH

        def chunk_body(g2, _, t=t, lo=lo, idx_hbm=idx_hbm):
            b = g2 // (OPW // OPC)
            g = g2 % (OPW // OPC)
            tb = t * B + b
            op0 = wid * OPW + g * OPC
            r0 = b * IDXR + op0
            pltpu.sync_copy(idx_hbm.at[pl.ds(r0, OPC)], idx_v)
            writes = [None, None]
            cp = gather(0, 0)
            for q in range(OPC):
                nxt = gather(q + 1, q + 1) if q + 1 < OPC else None
                cp.wait()
                if writes[q % 2] is not None:
                    writes[q % 2].wait()
                pkb = (q % 2) * GI
                for i in range(GI):
                    p0 = pk_v[pkb + i, pl.ds(lo, 16)]
                    p1 = pk_v[pkb + i, pl.ds(lo + 16, 16)]
                    pw = plsc.bitcast(
                        plsc.pack(p0, p1, format=plsc.PackFormat.INTERLEAVED),
                        jnp.float32,
                    )
                    pb_v[(q % 2) * ROWS_PER_OP + i // SPR,
                         pl.ds((i % SPR) * WPS, WPS)] = pw
                o0 = tb * GROWS2 + (op0 + q) * ROWS_PER_OP
                writes[q % 2] = pltpu.async_copy(
                    pb_v.at[pl.ds((q % 2) * ROWS_PER_OP, ROWS_PER_OP)],
                    g_hbm.at[pl.ds(o0, ROWS_PER_OP)],
                    sw[q % 2],
                )
                cp = nxt
            for w in writes:
                if w is not None:
                    w.wait()
            return 0

        lax.fori_loop(0, B * (OPW // OPC), chunk_body, 0, unroll=False)


@functools.cache
def _sc_gather():
    return pl.kernel(
        _sc_gather_body,
        out_type=jax.ShapeDtypeStruct((2 * B * GROWS2, LW), jnp.float32),
        mesh=plsc.VectorSubcoreMesh(core_axis_name="c", subcore_axis_name="s"),
        scratch_types=[
            pltpu.VMEM((OPC, GI), jnp.int32),
            pltpu.VMEM((2 * GI, LW), jnp.float32),
            pltpu.VMEM((2 * ROWS_PER_OP, LW), jnp.float32),
            pltpu.SemaphoreType.DMA,
            pltpu.SemaphoreType.DMA,
            pltpu.SemaphoreType.DMA,
            pltpu.SemaphoreType.DMA,
        ],
        compiler_params=pltpu.CompilerParams(needs_layout_passes=False),
    )


NB = 1024
GRID = (B * N) // NB
RPN = S // QO
_INV_SQRT2 = 0.7071067811865476


def _gelu(t):
    return 0.5 * t * (1.0 + lax.erf(t * _INV_SQRT2))


def _unpack_pair(u):
    lo = lax.bitcast_convert_type(u << 16, jnp.float32)
    hi = lax.bitcast_convert_type(u & jnp.int32(-65536), jnp.float32)
    return lo, hi


def _tc_combine_body(tp_ref, gj_ref, gk_ref, w2r_ref, b2_ref, out_ref):
    a = tp_ref[...][:, 2 * DH:3 * DH]
    a_lo = jnp.concatenate([a[:, :WPS]] * S, axis=1)
    a_hi = jnp.concatenate([a[:, WPS:]] * S, axis=1)
    ju = lax.bitcast_convert_type(gj_ref[...], jnp.int32)
    ku = lax.bitcast_convert_type(gk_ref[...], jnp.int32)
    j_lo, j_hi = _unpack_pair(ju)
    k_lo, k_hi = _unpack_pair(ku)
    h_lo = _gelu(j_lo + k_lo + a_lo)
    h_hi = _gelu(j_hi + k_hi + a_hi)
    h = jnp.concatenate([h_lo, h_hi], axis=1)
    o = jnp.dot(h, w2r_ref[...], preferred_element_type=jnp.float32)
    out_ref[...] = o + b2_ref[...]


_KOFF = (B * GROWS2) // NB

_tc_combine = pl.pallas_call(
    _tc_combine_body,
    grid=(GRID,),
    in_specs=[
        pl.BlockSpec((NB, LW), lambda i: (i, 0)),
        pl.BlockSpec((NB, LW), lambda i: (i, 0)),
        pl.BlockSpec((NB, LW), lambda i: (_KOFF + i, 0)),
        pl.BlockSpec((2 * LW, DOUT), lambda i: (0, 0)),
        pl.BlockSpec((1, DOUT), lambda i: (0, 0)),
    ],
    out_specs=pl.BlockSpec((NB, DOUT), lambda i: (i, 0)),
    out_shape=jax.ShapeDtypeStruct((B * N, DOUT), jnp.float32),
)


def kernel(x, j_idx, k_idx, W1, b1, W2, b2):
    off = (jnp.arange(B, dtype=jnp.int32) * N)[:, None, None]
    jr = (j_idx + off).reshape(B * IDXR, GI)
    kr = (k_idx + off).reshape(B * IDXR, GI)
    tp = _tc_project(x.reshape(B * N, D), W1, b1.reshape(1, DH))
    g = _sc_gather()(tp, jr, kr)
    w2r = jnp.concatenate(
        [W2[:WPS]] * S + [W2[WPS:]] * S, axis=0) / S
    out = _tc_combine(tp, g, g, w2r, b2.reshape(1, DOUT))
    return out.reshape(B, N, DOUT)


# speedup vs baseline: 1.9550x; 1.0044x over previous
import functools

import jax
import jax.numpy as jnp
from jax import lax
from jax.experimental import pallas as pl
from jax.experimental.pallas import tpu as pltpu
from jax.experimental.pallas import tpu_sc as plsc

B, N, D, S, DOUT = 2, 65536, 16, 8, 16
NS = N * S
DH = 2 * D
LW = 128

NBP = 4096


def _tc_project_body(x_ref, w1_ref, b1_ref, tp_ref):
    xa = x_ref[...]
    w1 = w1_ref[...]
    pj = jnp.dot(xa, w1[D:2 * D], preferred_element_type=jnp.float32)
    pk = jnp.dot(xa, w1[2 * D:], preferred_element_type=jnp.float32)
    pi = jnp.dot(xa, w1[:D], preferred_element_type=jnp.float32) + b1_ref[...]
    z = jnp.zeros((NBP, DH), jnp.float32)
    tp_ref[...] = jnp.concatenate([pj, pk, pi, z], axis=1)


_tc_project = pl.pallas_call(
    _tc_project_body,
    grid=((B * N) // NBP,),
    in_specs=[
        pl.BlockSpec((NBP, D), lambda i: (i, 0)),
        pl.BlockSpec((3 * D, DH), lambda i: (0, 0)),
        pl.BlockSpec((1, DH), lambda i: (0, 0)),
    ],
    out_specs=pl.BlockSpec((NBP, LW), lambda i: (i, 0)),
    out_shape=jax.ShapeDtypeStruct((B * N, LW), jnp.float32),
)

NC, NSUB = 2, 16
NW = NC * NSUB
PW = NS // NW
GI = 128
QO = 4
CO = GI * QO
CHUNKS = PW // CO
NCH = NS // CO
GROWS = NS // QO


IDXR = NS // GI
OPW = IDXR // NW
OPC = 8
SPR = S
ROWS_PER_OP = GI // SPR
GROWS2 = NS // SPR
WPS = DH // 2


def _sc_gather_body(tp_hbm, j_hbm, k_hbm, g_hbm, idx_v, pk_v, pb_v,
                    sg0, sg1, sw0, sw1):
    cid = lax.axis_index("c")
    sid = lax.axis_index("s")
    wid = sid * NC + cid
    sg = (sg0, sg1)
    sw = (sw0, sw1)

    def gather(q, row, b):
        return pltpu.async_copy(
            tp_hbm.at[pl.ds(b * N, N)].at[idx_v.at[row]],
            pk_v.at[pl.ds((q % 2) * GI, GI)],
            sg[q % 2],
        )

    for t, idx_hbm in ((0, j_hbm), (1, k_hbm)):
        lo = t * DH

        def chunk_body(g2, _, t=t, lo=lo, idx_hbm=idx_hbm):
            b = g2 // (OPW // OPC)
            g = g2 % (OPW // OPC)
            tb = t * B + b
            op0 = wid * OPW + g * OPC
            r0 = b * IDXR + op0
            pltpu.sync_copy(idx_hbm.at[pl.ds(r0, OPC)], idx_v)
            writes = [None, None]
            cp = gather(0, 0, b)
            for q in range(OPC):
                nxt = gather(q + 1, q + 1, b) if q + 1 < OPC else None
                cp.wait()
                if writes[q % 2] is not None:
                    writes[q % 2].wait()
                pkb = (q % 2) * GI
                for i in range(GI):
                    p0 = pk_v[pkb + i, pl.ds(lo, 16)]
                    p1 = pk_v[pkb + i, pl.ds(lo + 16, 16)]
                    pw = plsc.bitcast(
                        plsc.pack(p0, p1, format=plsc.PackFormat.INTERLEAVED),
                        jnp.float32,
                    )
                    pb_v[(q % 2) * ROWS_PER_OP + i // SPR,
                         pl.ds((i % SPR) * WPS, WPS)] = pw
                o0 = tb * GROWS2 + (op0 + q) * ROWS_PER_OP
                writes[q % 2] = pltpu.async_copy(
                    pb_v.at[pl.ds((q % 2) * ROWS_PER_OP, ROWS_PER_OP)],
                    g_hbm.at[pl.ds(o0, ROWS_PER_OP)],
                    sw[q % 2],
                )
                cp = nxt
            for w in writes:
                if w is not None:
                    w.wait()
            return 0

        lax.fori_loop(0, B * (OPW // OPC), chunk_body, 0, unroll=False)


@functools.cache
def _sc_gather():
    return pl.kernel(
        _sc_gather_body,
        out_type=jax.ShapeDtypeStruct((2 * B * GROWS2, LW), jnp.float32),
        mesh=plsc.VectorSubcoreMesh(core_axis_name="c", subcore_axis_name="s"),
        scratch_types=[
            pltpu.VMEM((OPC, GI), jnp.int32),
            pltpu.VMEM((2 * GI, LW), jnp.float32),
            pltpu.VMEM((2 * ROWS_PER_OP, LW), jnp.float32),
            pltpu.SemaphoreType.DMA,
            pltpu.SemaphoreType.DMA,
            pltpu.SemaphoreType.DMA,
            pltpu.SemaphoreType.DMA,
        ],
        compiler_params=pltpu.CompilerParams(needs_layout_passes=False),
    )


NB = 1024
GRID = (B * N) // NB
RPN = S // QO
_INV_SQRT2 = 0.7071067811865476


def _gelu(t):
    return 0.5 * t * (1.0 + lax.erf(t * _INV_SQRT2))


def _unpack_pair(u):
    lo = lax.bitcast_convert_type(u << 16, jnp.float32)
    hi = lax.bitcast_convert_type(u & jnp.int32(-65536), jnp.float32)
    return lo, hi


def _tc_combine_body(tp_ref, gj_ref, gk_ref, w2r_ref, b2_ref, out_ref):
    a = tp_ref[...][:, 2 * DH:3 * DH]
    a_lo = jnp.concatenate([a[:, :WPS]] * S, axis=1)
    a_hi = jnp.concatenate([a[:, WPS:]] * S, axis=1)
    ju = lax.bitcast_convert_type(gj_ref[...], jnp.int32)
    ku = lax.bitcast_convert_type(gk_ref[...], jnp.int32)
    j_lo, j_hi = _unpack_pair(ju)
    k_lo, k_hi = _unpack_pair(ku)
    h_lo = _gelu(j_lo + k_lo + a_lo)
    h_hi = _gelu(j_hi + k_hi + a_hi)
    h = jnp.concatenate([h_lo, h_hi], axis=1)
    o = jnp.dot(h, w2r_ref[...], preferred_element_type=jnp.float32)
    out_ref[...] = o + b2_ref[...]


_KOFF = (B * GROWS2) // NB

_tc_combine = pl.pallas_call(
    _tc_combine_body,
    grid=(GRID,),
    in_specs=[
        pl.BlockSpec((NB, LW), lambda i: (i, 0)),
        pl.BlockSpec((NB, LW), lambda i: (i, 0)),
        pl.BlockSpec((NB, LW), lambda i: (_KOFF + i, 0)),
        pl.BlockSpec((2 * LW, DOUT), lambda i: (0, 0)),
        pl.BlockSpec((1, DOUT), lambda i: (0, 0)),
    ],
    out_specs=pl.BlockSpec((NB, DOUT), lambda i: (i, 0)),
    out_shape=jax.ShapeDtypeStruct((B * N, DOUT), jnp.float32),
)


def kernel(x, j_idx, k_idx, W1, b1, W2, b2):
    jr = j_idx.reshape(B * IDXR, GI)
    kr = k_idx.reshape(B * IDXR, GI)
    tp = _tc_project(x.reshape(B * N, D), W1, b1.reshape(1, DH))
    g = _sc_gather()(tp, jr, kr)
    w2r = jnp.concatenate(
        [W2[:WPS]] * S + [W2[WPS:]] * S, axis=0) / S
    out = _tc_combine(tp, g, g, w2r, b2.reshape(1, DOUT))
    return out.reshape(B, N, DOUT)


# speedup vs baseline: 2.0515x; 1.0494x over previous
import functools

import jax
import jax.numpy as jnp
from jax import lax
from jax.experimental import pallas as pl
from jax.experimental.pallas import tpu as pltpu
from jax.experimental.pallas import tpu_sc as plsc

B, N, D, S, DOUT = 2, 65536, 16, 8, 16
NS = N * S
DH = 2 * D
LW = 128

NBP = 4096


def _tc_project_body(x_ref, w1_ref, b1_ref, tp_ref):
    xa = x_ref[...]
    w1 = w1_ref[...]
    pj = jnp.dot(xa, w1[D:2 * D], preferred_element_type=jnp.float32)
    pk = jnp.dot(xa, w1[2 * D:], preferred_element_type=jnp.float32)
    pi = jnp.dot(xa, w1[:D], preferred_element_type=jnp.float32) + b1_ref[...]
    z = jnp.zeros((NBP, DH), jnp.float32)
    tp_ref[...] = jnp.concatenate([pj, pk, pi, z], axis=1)


_tc_project = pl.pallas_call(
    _tc_project_body,
    grid=((B * N) // NBP,),
    in_specs=[
        pl.BlockSpec((NBP, D), lambda i: (i, 0)),
        pl.BlockSpec((3 * D, DH), lambda i: (0, 0)),
        pl.BlockSpec((1, DH), lambda i: (0, 0)),
    ],
    out_specs=pl.BlockSpec((NBP, LW), lambda i: (i, 0)),
    out_shape=jax.ShapeDtypeStruct((B * N, LW), jnp.float32),
)

NC, NSUB = 2, 16
NW = NC * NSUB
PW = NS // NW
GI = 128
QO = 4
CO = GI * QO
CHUNKS = PW // CO
NCH = NS // CO
GROWS = NS // QO


IDXR = NS // GI
OPW = IDXR // NW
OPC = 8
SPR = S
ROWS_PER_OP = GI // SPR
GROWS2 = NS // SPR
WPS = DH // 2


def _sc_gather_body(tp_hbm, j_hbm, k_hbm, g_hbm, idx_v, pk_v, pb_v,
                    sg0, sg1, sw0, sw1, si):
    cid = lax.axis_index("c")
    sid = lax.axis_index("s")
    wid = sid * NC + cid
    sg = (sg0, sg1)
    sw = (sw0, sw1)

    def gather(q, row, b):
        return pltpu.async_copy(
            tp_hbm.at[pl.ds(b * N, N)].at[idx_v.at[row]],
            pk_v.at[pl.ds((q % 2) * GI, GI)],
            sg[q % 2],
        )

    CPG = OPW // OPC

    for t, idx_hbm in ((0, j_hbm), (1, k_hbm)):
        lo = t * DH

        def idx_rows(g2):
            b = g2 // CPG
            g = g2 % CPG
            return b * IDXR + wid * OPW + g * OPC

        pltpu.async_copy(
            idx_hbm.at[pl.ds(idx_rows(0), OPC)],
            idx_v.at[pl.ds(0, OPC)], si,
        )

        def chunk_body(g2, _, t=t, lo=lo, idx_hbm=idx_hbm):
            b = g2 // CPG
            g = g2 % CPG
            tb = t * B + b
            op0 = wid * OPW + g * OPC
            par = (g2 % 2) * OPC
            pltpu.make_async_copy(
                idx_hbm.at[pl.ds(0, OPC)], idx_v.at[pl.ds(par, OPC)], si
            ).wait()
            g2n = jnp.minimum(g2 + 1, B * CPG - 1)
            pltpu.async_copy(
                idx_hbm.at[pl.ds(idx_rows(g2n), OPC)],
                idx_v.at[pl.ds(OPC - par, OPC)], si,
            )
            writes = [None, None]
            cp = gather(0, par + 0, b)
            for q in range(OPC):
                nxt = gather(q + 1, par + q + 1, b) if q + 1 < OPC else None
                cp.wait()
                if writes[q % 2] is not None:
                    writes[q % 2].wait()
                pkb = (q % 2) * GI
                for i in range(GI):
                    p0 = pk_v[pkb + i, pl.ds(lo, 16)]
                    p1 = pk_v[pkb + i, pl.ds(lo + 16, 16)]
                    pw = plsc.bitcast(
                        plsc.pack(p0, p1, format=plsc.PackFormat.INTERLEAVED),
                        jnp.float32,
                    )
                    pb_v[(q % 2) * ROWS_PER_OP + i // SPR,
                         pl.ds((i % SPR) * WPS, WPS)] = pw
                o0 = tb * GROWS2 + (op0 + q) * ROWS_PER_OP
                writes[q % 2] = pltpu.async_copy(
                    pb_v.at[pl.ds((q % 2) * ROWS_PER_OP, ROWS_PER_OP)],
                    g_hbm.at[pl.ds(o0, ROWS_PER_OP)],
                    sw[q % 2],
                )
                cp = nxt
            for w in writes:
                if w is not None:
                    w.wait()
            return 0

        lax.fori_loop(0, B * CPG, chunk_body, 0, unroll=False)
        pltpu.make_async_copy(
            idx_hbm.at[pl.ds(0, OPC)], idx_v.at[pl.ds(0, OPC)], si
        ).wait()


@functools.cache
def _sc_gather():
    return pl.kernel(
        _sc_gather_body,
        out_type=jax.ShapeDtypeStruct((2 * B * GROWS2, LW), jnp.float32),
        mesh=plsc.VectorSubcoreMesh(core_axis_name="c", subcore_axis_name="s"),
        scratch_types=[
            pltpu.VMEM((2 * OPC, GI), jnp.int32),
            pltpu.VMEM((2 * GI, LW), jnp.float32),
            pltpu.VMEM((2 * ROWS_PER_OP, LW), jnp.float32),
            pltpu.SemaphoreType.DMA,
            pltpu.SemaphoreType.DMA,
            pltpu.SemaphoreType.DMA,
            pltpu.SemaphoreType.DMA,
            pltpu.SemaphoreType.DMA,
        ],
        compiler_params=pltpu.CompilerParams(needs_layout_passes=False),
    )


NB = 1024
GRID = (B * N) // NB
RPN = S // QO
_INV_SQRT2 = 0.7071067811865476


def _gelu(t):
    return 0.5 * t * (1.0 + lax.erf(t * _INV_SQRT2))


def _unpack_pair(u):
    lo = lax.bitcast_convert_type(u << 16, jnp.float32)
    hi = lax.bitcast_convert_type(u & jnp.int32(-65536), jnp.float32)
    return lo, hi


def _tc_combine_body(tp_ref, gj_ref, gk_ref, w2r_ref, b2_ref, out_ref):
    a = tp_ref[...][:, 2 * DH:3 * DH]
    a_lo = jnp.concatenate([a[:, :WPS]] * S, axis=1)
    a_hi = jnp.concatenate([a[:, WPS:]] * S, axis=1)
    ju = lax.bitcast_convert_type(gj_ref[...], jnp.int32)
    ku = lax.bitcast_convert_type(gk_ref[...], jnp.int32)
    j_lo, j_hi = _unpack_pair(ju)
    k_lo, k_hi = _unpack_pair(ku)
    h_lo = _gelu(j_lo + k_lo + a_lo)
    h_hi = _gelu(j_hi + k_hi + a_hi)
    h = jnp.concatenate([h_lo, h_hi], axis=1)
    o = jnp.dot(h, w2r_ref[...], preferred_element_type=jnp.float32)
    out_ref[...] = o + b2_ref[...]


_KOFF = (B * GROWS2) // NB

_tc_combine = pl.pallas_call(
    _tc_combine_body,
    grid=(GRID,),
    in_specs=[
        pl.BlockSpec((NB, LW), lambda i: (i, 0)),
        pl.BlockSpec((NB, LW), lambda i: (i, 0)),
        pl.BlockSpec((NB, LW), lambda i: (_KOFF + i, 0)),
        pl.BlockSpec((2 * LW, DOUT), lambda i: (0, 0)),
        pl.BlockSpec((1, DOUT), lambda i: (0, 0)),
    ],
    out_specs=pl.BlockSpec((NB, DOUT), lambda i: (i, 0)),
    out_shape=jax.ShapeDtypeStruct((B * N, DOUT), jnp.float32),
)


def kernel(x, j_idx, k_idx, W1, b1, W2, b2):
    jr = j_idx.reshape(B * IDXR, GI)
    kr = k_idx.reshape(B * IDXR, GI)
    tp = _tc_project(x.reshape(B * N, D), W1, b1.reshape(1, DH))
    g = _sc_gather()(tp, jr, kr)
    w2r = jnp.concatenate(
        [W2[:WPS]] * S + [W2[WPS:]] * S, axis=0) / S
    out = _tc_combine(tp, g, g, w2r, b2.reshape(1, DOUT))
    return out.reshape(B, N, DOUT)


# speedup vs baseline: 2.0695x; 1.0087x over previous
import functools

import jax
import jax.numpy as jnp
from jax import lax
from jax.experimental import pallas as pl
from jax.experimental.pallas import tpu as pltpu
from jax.experimental.pallas import tpu_sc as plsc

B, N, D, S, DOUT = 2, 65536, 16, 8, 16
NS = N * S
DH = 2 * D
LW = 128

NBP = 4096


def _tc_project_body(x_ref, w1_ref, b1_ref, tp_ref):
    xa = x_ref[...]
    w1 = w1_ref[...]
    pj = jnp.dot(xa, w1[D:2 * D], preferred_element_type=jnp.float32)
    pk = jnp.dot(xa, w1[2 * D:], preferred_element_type=jnp.float32)
    pi = jnp.dot(xa, w1[:D], preferred_element_type=jnp.float32) + b1_ref[...]
    z = jnp.zeros((NBP, DH), jnp.float32)
    tp_ref[...] = jnp.concatenate([pj, pk, pi, z], axis=1)


_tc_project = pl.pallas_call(
    _tc_project_body,
    grid=((B * N) // NBP,),
    in_specs=[
        pl.BlockSpec((NBP, D), lambda i: (i, 0)),
        pl.BlockSpec((3 * D, DH), lambda i: (0, 0)),
        pl.BlockSpec((1, DH), lambda i: (0, 0)),
    ],
    out_specs=pl.BlockSpec((NBP, LW), lambda i: (i, 0)),
    out_shape=jax.ShapeDtypeStruct((B * N, LW), jnp.float32),
)

NC, NSUB = 2, 16
NW = NC * NSUB
PW = NS // NW
GI = 128
QO = 4
CO = GI * QO
CHUNKS = PW // CO
NCH = NS // CO
GROWS = NS // QO


IDXR = NS // GI
OPW = IDXR // NW
OPC = 8
SPR = S
ROWS_PER_OP = GI // SPR
GROWS2 = NS // SPR
WPS = DH // 2


def _sc_gather_body(tp_hbm, j_hbm, k_hbm, g_hbm, idx_v, pk_v, pb_v,
                    sg0, sg1, sw0, sw1, si):
    cid = lax.axis_index("c")
    sid = lax.axis_index("s")
    wid = sid * NC + cid
    sg = (sg0, sg1)
    sw = (sw0, sw1)

    def gather(q, row, b):
        return pltpu.async_copy(
            tp_hbm.at[pl.ds(b * N, N)].at[idx_v.at[row]],
            pk_v.at[pl.ds((q % 2) * GI, GI)],
            sg[q % 2],
        )

    CPG = OPW // OPC

    for t, idx_hbm in ((0, j_hbm), (1, k_hbm)):
        lo = t * DH

        def idx_rows(g2):
            b = g2 // CPG
            g = g2 % CPG
            return b * IDXR + wid * OPW + g * OPC

        pltpu.async_copy(
            idx_hbm.at[pl.ds(idx_rows(0), OPC)],
            idx_v.at[pl.ds(0, OPC)], si,
        )

        def chunk_body(g2, _, t=t, lo=lo, idx_hbm=idx_hbm):
            b = g2 // CPG
            g = g2 % CPG
            tb = t * B + b
            op0 = wid * OPW + g * OPC
            par = (g2 % 2) * OPC
            pltpu.make_async_copy(
                idx_hbm.at[pl.ds(0, OPC)], idx_v.at[pl.ds(par, OPC)], si
            ).wait()
            g2n = jnp.minimum(g2 + 1, B * CPG - 1)
            pltpu.async_copy(
                idx_hbm.at[pl.ds(idx_rows(g2n), OPC)],
                idx_v.at[pl.ds(OPC - par, OPC)], si,
            )
            writes = [None, None]
            cp = gather(0, par + 0, b)
            for q in range(OPC):
                nxt = gather(q + 1, par + q + 1, b) if q + 1 < OPC else None
                cp.wait()
                if writes[q % 2] is not None:
                    writes[q % 2].wait()
                pkb = (q % 2) * GI
                for i in range(GI):
                    p0 = pk_v[pkb + i, pl.ds(lo, 16)]
                    p1 = pk_v[pkb + i, pl.ds(lo + 16, 16)]
                    pw = plsc.bitcast(
                        plsc.pack(p0, p1, format=plsc.PackFormat.INTERLEAVED),
                        jnp.float32,
                    )
                    pb_v[(q % 2) * ROWS_PER_OP + i // SPR,
                         pl.ds((i % SPR) * WPS, WPS)] = pw
                o0 = tb * GROWS2 + (op0 + q) * ROWS_PER_OP
                writes[q % 2] = pltpu.async_copy(
                    pb_v.at[pl.ds((q % 2) * ROWS_PER_OP, ROWS_PER_OP)],
                    g_hbm.at[pl.ds(o0, ROWS_PER_OP)],
                    sw[q % 2],
                )
                cp = nxt
            for w in writes:
                if w is not None:
                    w.wait()
            return 0

        lax.fori_loop(0, B * CPG, chunk_body, 0, unroll=False)
        pltpu.make_async_copy(
            idx_hbm.at[pl.ds(0, OPC)], idx_v.at[pl.ds(0, OPC)], si
        ).wait()


@functools.cache
def _sc_gather():
    return pl.kernel(
        _sc_gather_body,
        out_type=jax.ShapeDtypeStruct((2 * B * GROWS2, LW), jnp.float32),
        mesh=plsc.VectorSubcoreMesh(core_axis_name="c", subcore_axis_name="s"),
        scratch_types=[
            pltpu.VMEM((2 * OPC, GI), jnp.int32),
            pltpu.VMEM((2 * GI, LW), jnp.float32),
            pltpu.VMEM((2 * ROWS_PER_OP, LW), jnp.float32),
            pltpu.SemaphoreType.DMA,
            pltpu.SemaphoreType.DMA,
            pltpu.SemaphoreType.DMA,
            pltpu.SemaphoreType.DMA,
            pltpu.SemaphoreType.DMA,
        ],
        compiler_params=pltpu.CompilerParams(needs_layout_passes=False),
    )


NB = 4096
GRID = (B * N) // NB
RPN = S // QO
_INV_SQRT2 = 0.7071067811865476


def _gelu(t):
    return 0.5 * t * (1.0 + lax.erf(t * _INV_SQRT2))


def _unpack_pair(u):
    lo = lax.bitcast_convert_type(u << 16, jnp.float32)
    hi = lax.bitcast_convert_type(u & jnp.int32(-65536), jnp.float32)
    return lo, hi


def _tc_combine_body(tp_ref, gj_ref, gk_ref, w2r_ref, b2_ref, out_ref):
    a = tp_ref[...][:, 2 * DH:3 * DH]
    a_lo = jnp.concatenate([a[:, :WPS]] * S, axis=1)
    a_hi = jnp.concatenate([a[:, WPS:]] * S, axis=1)
    ju = lax.bitcast_convert_type(gj_ref[...], jnp.int32)
    ku = lax.bitcast_convert_type(gk_ref[...], jnp.int32)
    j_lo, j_hi = _unpack_pair(ju)
    k_lo, k_hi = _unpack_pair(ku)
    h_lo = _gelu(j_lo + k_lo + a_lo)
    h_hi = _gelu(j_hi + k_hi + a_hi)
    h = jnp.concatenate([h_lo, h_hi], axis=1)
    o = jnp.dot(h, w2r_ref[...], preferred_element_type=jnp.float32)
    out_ref[...] = o + b2_ref[...]


_KOFF = (B * GROWS2) // NB

_tc_combine = pl.pallas_call(
    _tc_combine_body,
    grid=(GRID,),
    in_specs=[
        pl.BlockSpec((NB, LW), lambda i: (i, 0)),
        pl.BlockSpec((NB, LW), lambda i: (i, 0)),
        pl.BlockSpec((NB, LW), lambda i: (_KOFF + i, 0)),
        pl.BlockSpec((2 * LW, DOUT), lambda i: (0, 0)),
        pl.BlockSpec((1, DOUT), lambda i: (0, 0)),
    ],
    out_specs=pl.BlockSpec((NB, DOUT), lambda i: (i, 0)),
    out_shape=jax.ShapeDtypeStruct((B * N, DOUT), jnp.float32),
)


def kernel(x, j_idx, k_idx, W1, b1, W2, b2):
    jr = j_idx.reshape(B * IDXR, GI)
    kr = k_idx.reshape(B * IDXR, GI)
    tp = _tc_project(x.reshape(B * N, D), W1, b1.reshape(1, DH))
    g = _sc_gather()(tp, jr, kr)
    w2r = jnp.concatenate(
        [W2[:WPS]] * S + [W2[WPS:]] * S, axis=0) / S
    out = _tc_combine(tp, g, g, w2r, b2.reshape(1, DOUT))
    return out.reshape(B, N, DOUT)
